# baseline (speedup 1.0000x reference)
import functools

import jax
import jax.numpy as jnp
from jax import lax
from jax.experimental import pallas as pl
from jax.experimental.pallas import tpu as pltpu
from jax.experimental.pallas import tpu_sc as plsc

N = 10000
D = 128
G = 20
MAXN = 500
LAT = 64
E = 320000

NUM_CORES = 2
NUM_SUBCORES = 16
NUM_TILES = NUM_CORES * NUM_SUBCORES

ROWS_PER_TILE = 632
NPAD = NUM_SUBCORES * ROWS_PER_TILE
SRC_FILL = NPAD - 2
DST_FILL = NPAD - 1

CHUNK = 128
CHUNKS_PER_TILE = 80
EDGES_PER_TILE = CHUNKS_PER_TILE * CHUNK
EPAD = NUM_TILES * EDGES_PER_TILE
NCHUNKS = EPAD // CHUNK

NBUF = 4
BLKCH = 8
NBLOCKS = CHUNKS_PER_TILE // BLKCH

GPOOL = 32


def _sc_mesh():
    return plsc.VectorSubcoreMesh(core_axis_name="c", subcore_axis_name="s")


def _sc_count(edges, zeros_cnt, ones_blk):
    @functools.partial(
        pl.kernel,
        out_type=jax.ShapeDtypeStruct((NUM_CORES, NPAD, D), jnp.float32),
        mesh=_sc_mesh(),
        scratch_types=[
            pltpu.VMEM_SHARED((NPAD, D), jnp.float32),
            pltpu.VMEM((CHUNKS_PER_TILE, 2, CHUNK), jnp.int32),
            pltpu.VMEM((CHUNK, D), jnp.float32),
        ] + [pltpu.SemaphoreType.DMA] * NBUF,
    )
    def k(edges_hbm, zeros_hbm, ones_hbm, out_hbm, acc_sh, idx_v, ones_v,
          *sem_s):
        c = lax.axis_index("c")
        s = lax.axis_index("s")
        row0 = s * ROWS_PER_TILE
        rows = pl.ds(row0, ROWS_PER_TILE)
        pltpu.sync_copy(zeros_hbm.at[rows], acc_sh.at[rows])
        pltpu.sync_copy(ones_hbm, ones_v)
        chunk0 = (c * NUM_SUBCORES + s) * CHUNKS_PER_TILE
        pltpu.sync_copy(edges_hbm.at[pl.ds(chunk0, CHUNKS_PER_TILE)], idx_v)
        plsc.subcore_barrier()

        for b in range(NBUF):
            pltpu.async_copy(ones_v, acc_sh.at[idx_v.at[b].at[1]], sem_s[b],
                             add=True)

        @pl.loop(0, CHUNKS_PER_TILE // NBUF - 1)
        def _(t):
            j0 = t * NBUF + NBUF
            for b in range(NBUF):
                pltpu.make_async_copy(zeros_hbm.at[pl.ds(0, CHUNK)], ones_v,
                                      sem_s[b]).wait()
                pltpu.async_copy(ones_v, acc_sh.at[idx_v.at[j0 + b].at[1]],
                                 sem_s[b], add=True)

        for b in range(NBUF):
            pltpu.make_async_copy(zeros_hbm.at[pl.ds(0, CHUNK)], ones_v,
                                  sem_s[b]).wait()

        plsc.subcore_barrier()
        pltpu.sync_copy(acc_sh.at[rows], out_hbm.at[c].at[rows])

    return k(edges, zeros_cnt, ones_blk)


def _sc_aggregate(h, zeros_f, edges):
    @functools.partial(
        pl.kernel,
        out_type=jax.ShapeDtypeStruct((NUM_CORES, NPAD, D), jnp.float32),
        mesh=_sc_mesh(),
        scratch_types=[
            pltpu.VMEM_SHARED((NPAD, D), jnp.float32),
            pltpu.VMEM((2, BLKCH, 2, CHUNK), jnp.int32),
            pltpu.VMEM((2, CHUNK, D), jnp.float32),
        ] + [pltpu.SemaphoreType.DMA] * 6,
    )
    def k(h_hbm, z_hbm, edges_hbm, out_hbm, acc_sh, idx_v, rows_v,
          sem_i0, sem_i1, sem_g0, sem_g1, sem_s0, sem_s1):
        sem_i = (sem_i0, sem_i1)
        sem_g = (sem_g0, sem_g1)
        sem_s = (sem_s0, sem_s1)
        c = lax.axis_index("c")
        s = lax.axis_index("s")
        row0 = s * ROWS_PER_TILE
        rows = pl.ds(row0, ROWS_PER_TILE)

        @pl.when(c == 0)
        def _():
            pltpu.sync_copy(h_hbm.at[rows], acc_sh.at[rows])

        @pl.when(c != 0)
        def _():
            pltpu.sync_copy(z_hbm.at[rows], acc_sh.at[rows])

        chunk0 = (c * NUM_SUBCORES + s) * CHUNKS_PER_TILE

        def wait_idx(kb):
            pltpu.make_async_copy(edges_hbm.at[pl.ds(0, BLKCH)],
                                  idx_v.at[kb], sem_i[kb]).wait()

        def wait_rows(sem):
            pltpu.make_async_copy(h_hbm.at[pl.ds(0, CHUNK)],
                                  rows_v.at[0], sem).wait()

        def fire_gather(kb, jj, slot):
            pltpu.async_copy(h_hbm.at[idx_v.at[kb].at[jj].at[0]],
                             rows_v.at[slot], sem_g[slot])

        def fire_scatter(kb, jj, slot):
            pltpu.async_copy(rows_v.at[slot],
                             acc_sh.at[idx_v.at[kb].at[jj].at[1]],
                             sem_s[slot], add=True)

        def do_pair(kb, jj, last_blk=False):
            wait_rows(sem_g[0])
            fire_scatter(kb, jj, 0)
            wait_rows(sem_g[1])
            fire_scatter(kb, jj + 1, 1)
            if jj == 6 and not last_blk:
                wait_idx(1 - kb)
            wait_rows(sem_s[0])
            if not (last_blk and jj == 6):
                if jj < 6:
                    fire_gather(kb, jj + 2, 0)
                else:
                    fire_gather(1 - kb, 0, 0)
            wait_rows(sem_s[1])
            if not (last_blk and jj == 6):
                if jj < 6:
                    fire_gather(kb, jj + 3, 1)
                else:
                    fire_gather(1 - kb, 1, 1)

        pltpu.sync_copy(edges_hbm.at[pl.ds(chunk0, BLKCH)], idx_v.at[0])
        pltpu.async_copy(edges_hbm.at[pl.ds(chunk0 + BLKCH, BLKCH)],
                         idx_v.at[1], sem_i[1])
        plsc.subcore_barrier()
        fire_gather(0, 0, 0)
        fire_gather(0, 1, 1)

        @pl.loop(0, (NBLOCKS - 2) // 2)
        def _(p):
            bA = 2 * p
            for jj in range(0, BLKCH, 2):
                do_pair(0, jj)
            pltpu.async_copy(
                edges_hbm.at[pl.ds(chunk0 + (bA + 2) * BLKCH, BLKCH)],
                idx_v.at[0], sem_i[0])
            for jj in range(0, BLKCH, 2):
                do_pair(1, jj)
            pltpu.async_copy(
                edges_hbm.at[pl.ds(chunk0 + (bA + 3) * BLKCH, BLKCH)],
                idx_v.at[1], sem_i[1])

        for jj in range(0, BLKCH, 2):
            do_pair(0, jj)
        for jj in range(0, BLKCH, 2):
            do_pair(1, jj, last_blk=True)

        plsc.subcore_barrier()
        pltpu.sync_copy(acc_sh.at[rows], out_hbm.at[c].at[rows])

    return k(h, zeros_f, edges)


def _tc_matmul(x, W):
    def body(x_ref, w_ref, o_ref):
        o_ref[...] = jnp.dot(x_ref[...], w_ref[...],
                             preferred_element_type=jnp.float32)

    return pl.pallas_call(
        body,
        out_shape=jax.ShapeDtypeStruct((x.shape[0], W.shape[1]), jnp.float32),
    )(x, W)


def _tc_scale(cnt, hw):
    def body(cnt_ref, hw_ref, dinv_ref, h_ref):
        deg = 1.0 + cnt_ref[0, :, 0:1] + cnt_ref[1, :, 0:1]
        dinv = lax.rsqrt(deg)
        dinv_ref[...] = dinv
        f = hw_ref.shape[1]
        h_ref[:, 0:f] = dinv * hw_ref[...]
        h_ref[:, f:D] = jnp.zeros((NPAD, D - f), jnp.float32)

    return pl.pallas_call(
        body,
        out_shape=(
            jax.ShapeDtypeStruct((NPAD, 1), jnp.float32),
            jax.ShapeDtypeStruct((NPAD, D), jnp.float32),
        ),
    )(cnt, hw)


def _tc_layer(a, dinv, b, W):
    fin = W.shape[0]
    fout = W.shape[1]

    def body(a_ref, dinv_ref, b_ref, w_ref, o_ref):
        h = a_ref[0, :, 0:fin] + a_ref[1, :, 0:fin]
        h = jnp.maximum(dinv_ref[...] * h + b_ref[...], 0.0)
        o_ref[:, 0:fout] = dinv_ref[...] * jnp.dot(
            h, w_ref[...], preferred_element_type=jnp.float32)
        if fout < D:
            o_ref[:, fout:D] = jnp.zeros((NPAD, D - fout), jnp.float32)

    return pl.pallas_call(
        body,
        out_shape=jax.ShapeDtypeStruct((NPAD, D), jnp.float32),
    )(a, dinv, b, W)


def _tc_head(a, dinv, b3, batch2, Wmu, bmu, Wlv, blv, eps, Wd1, bd1, gamma,
             beta, Wd2, bd2):
    def body(a_ref, dinv_ref, b3_ref, batch_ref, wmu_ref, bmu_ref, wlv_ref,
             blv_ref, eps_ref, wd1_ref, bd1_ref, g_ref, be_ref, wd2_ref,
             bd2_ref, mu_ref, lv_ref, d2_ref):
        h = a_ref[0] + a_ref[1]
        h = jnp.maximum(dinv_ref[...] * h + b3_ref[...], 0.0)
        gids = lax.broadcasted_iota(jnp.int32, (GPOOL, NPAD), 0)
        mask = (gids == batch_ref[...]).astype(jnp.float32)
        sums = jnp.dot(mask, h, preferred_element_type=jnp.float32)
        cnt = jnp.sum(mask, axis=1, keepdims=True)
        pooled = (sums / jnp.maximum(cnt, 1.0))[0:G]
        mu = jnp.dot(pooled, wmu_ref[...],
                     preferred_element_type=jnp.float32) + bmu_ref[...]
        lv = jnp.dot(pooled, wlv_ref[...],
                     preferred_element_type=jnp.float32) + blv_ref[...]
        mu_ref[...] = mu
        lv_ref[...] = lv
        z = mu + eps_ref[...] * jnp.exp(0.5 * lv)
        d = jnp.maximum(jnp.dot(z, wd1_ref[...],
                                preferred_element_type=jnp.float32)
                        + bd1_ref[...], 0.0)
        m = jnp.mean(d, axis=0, keepdims=True)
        v = jnp.mean((d - m) * (d - m), axis=0, keepdims=True)
        dn = (d - m) / jnp.sqrt(v + 1e-5) * g_ref[...] + be_ref[...]
        dn = jnp.maximum(dn, 0.0)
        d2_ref[...] = jnp.dot(dn, wd2_ref[...],
                              preferred_element_type=jnp.float32) + bd2_ref[...]

    return pl.pallas_call(
        body,
        out_shape=(
            jax.ShapeDtypeStruct((G, LAT), jnp.float32),
            jax.ShapeDtypeStruct((G, LAT), jnp.float32),
            jax.ShapeDtypeStruct((G, MAXN * 32), jnp.float32),
        ),
    )(a, dinv, b3, batch2, Wmu, bmu, Wlv, blv, eps, Wd1, bd1, gamma, beta,
      Wd2, bd2)


def _tc_adj(nr):
    def body(nr_ref, o_ref):
        v = nr_ref[0]
        a = lax.dot_general(v, v, dimension_numbers=(((1,), (1,)), ((), ())),
                            preferred_element_type=jnp.float32)
        r = lax.broadcasted_iota(jnp.int32, (MAXN, MAXN), 0)
        cc = lax.broadcasted_iota(jnp.int32, (MAXN, MAXN), 1)
        a = jnp.where(r == cc, 0.0, a)
        o_ref[0] = 1.0 / (1.0 + jnp.exp(-a))

    return pl.pallas_call(
        body,
        grid=(G,),
        in_specs=[pl.BlockSpec((1, MAXN, 32), lambda g: (g, 0, 0))],
        out_specs=pl.BlockSpec((1, MAXN, MAXN), lambda g: (g, 0, 0)),
        out_shape=jax.ShapeDtypeStruct((G, MAXN, MAXN), jnp.float32),
    )(nr)


def kernel(x, edge_index, batch, W1, b1, W2, b2, W3, b3, Wmu, bmu, Wlv, blv,
           Wd1, bd1, gamma, beta, Wd2, bd2):
    f32 = jnp.float32
    xp = jnp.pad(x, ((0, NPAD - N), (0, 0)))
    srcp = jnp.pad(edge_index[0], (0, EPAD - E), constant_values=SRC_FILL)
    dstp = jnp.pad(edge_index[1], (0, EPAD - E), constant_values=DST_FILL)
    edges = jnp.stack([srcp.reshape(NCHUNKS, CHUNK),
                       dstp.reshape(NCHUNKS, CHUNK)], axis=1)

    zeros_cnt = jnp.zeros((NPAD, D), f32)
    ones_blk = jnp.ones((CHUNK, D), f32)
    z128 = jnp.zeros((NPAD, D), f32)

    cnt = _sc_count(edges, zeros_cnt, ones_blk)
    hw1 = _tc_matmul(xp, W1)
    dinv, h1p = _tc_scale(cnt, hw1)

    a1 = _sc_aggregate(h1p, z128, edges)
    h2p = _tc_layer(a1, dinv, b1.reshape(1, -1), W2)
    a2 = _sc_aggregate(h2p, z128, edges)
    h3p = _tc_layer(a2, dinv, b2.reshape(1, -1), W3)
    a3 = _sc_aggregate(h3p, z128, edges)

    eps = jax.random.normal(jax.random.key(42), (G, LAT), f32)
    batch2 = jnp.pad(batch, (0, NPAD - N), constant_values=G).reshape(1, NPAD)
    mu, logvar, d2 = _tc_head(
        a3, dinv, b3.reshape(1, -1), batch2, Wmu, bmu.reshape(1, -1), Wlv,
        blv.reshape(1, -1), eps, Wd1, bd1.reshape(1, -1),
        gamma.reshape(1, -1), beta.reshape(1, -1), Wd2, bd2.reshape(1, -1))

    nr = d2.reshape(G, MAXN, 32)
    adj = _tc_adj(nr)
    return adj, mu, logvar


# speedup vs baseline: 2.7130x; 2.7130x over previous
import functools

import jax
import jax.numpy as jnp
from jax import lax
from jax.experimental import pallas as pl
from jax.experimental.pallas import tpu as pltpu
from jax.experimental.pallas import tpu_sc as plsc

N = 10000
D = 128
G = 20
MAXN = 500
LAT = 64
E = 320000

NUM_CORES = 2
NUM_SUBCORES = 16
NUM_TILES = NUM_CORES * NUM_SUBCORES

ROWS_PER_TILE = 632
NPAD = NUM_SUBCORES * ROWS_PER_TILE
SRC_FILL = NPAD - 2
DST_FILL = NPAD - 1

CHUNK = 128
CHUNKS_PER_TILE = 80
EDGES_PER_TILE = CHUNKS_PER_TILE * CHUNK
EPAD = NUM_TILES * EDGES_PER_TILE
NCHUNKS = EPAD // CHUNK

NBUF = 4
BLKCH = 8
NBLOCKS = CHUNKS_PER_TILE // BLKCH

GPOOL = 32


def _sc_mesh():
    return plsc.VectorSubcoreMesh(core_axis_name="c", subcore_axis_name="s")


def _sc_count(edges, zeros_cnt, ones_blk):
    @functools.partial(
        pl.kernel,
        out_type=jax.ShapeDtypeStruct((NUM_CORES, NPAD, D), jnp.float32),
        mesh=_sc_mesh(),
        scratch_types=[
            pltpu.VMEM_SHARED((NPAD, D), jnp.float32),
            pltpu.VMEM((CHUNKS_PER_TILE, 2, CHUNK), jnp.int32),
            pltpu.VMEM((CHUNK, D), jnp.float32),
        ] + [pltpu.SemaphoreType.DMA] * NBUF,
    )
    def k(edges_hbm, zeros_hbm, ones_hbm, out_hbm, acc_sh, idx_v, ones_v,
          *sem_s):
        c = lax.axis_index("c")
        s = lax.axis_index("s")
        row0 = s * ROWS_PER_TILE
        rows = pl.ds(row0, ROWS_PER_TILE)
        pltpu.sync_copy(zeros_hbm.at[rows], acc_sh.at[rows])
        pltpu.sync_copy(ones_hbm, ones_v)
        chunk0 = (c * NUM_SUBCORES + s) * CHUNKS_PER_TILE
        pltpu.sync_copy(edges_hbm.at[pl.ds(chunk0, CHUNKS_PER_TILE)], idx_v)
        plsc.subcore_barrier()

        for b in range(NBUF):
            pltpu.async_copy(ones_v, acc_sh.at[idx_v.at[b].at[1]], sem_s[b],
                             add=True)

        @pl.loop(0, CHUNKS_PER_TILE // NBUF - 1)
        def _(t):
            j0 = t * NBUF + NBUF
            for b in range(NBUF):
                pltpu.make_async_copy(zeros_hbm.at[pl.ds(0, CHUNK)], ones_v,
                                      sem_s[b]).wait()
                pltpu.async_copy(ones_v, acc_sh.at[idx_v.at[j0 + b].at[1]],
                                 sem_s[b], add=True)

        for b in range(NBUF):
            pltpu.make_async_copy(zeros_hbm.at[pl.ds(0, CHUNK)], ones_v,
                                  sem_s[b]).wait()

        plsc.subcore_barrier()
        pltpu.sync_copy(acc_sh.at[rows], out_hbm.at[c].at[rows])

    return k(edges, zeros_cnt, ones_blk)


def _sc_aggregate(h, zeros_f, edges):
    @functools.partial(
        pl.kernel,
        out_type=jax.ShapeDtypeStruct((NUM_CORES, NPAD, D), jnp.float32),
        mesh=_sc_mesh(),
        scratch_types=[
            pltpu.VMEM_SHARED((NPAD, D), jnp.float32),
            pltpu.VMEM((2, BLKCH, 2, CHUNK), jnp.int32),
            pltpu.VMEM((2, CHUNK, D), jnp.float32),
        ] + [pltpu.SemaphoreType.DMA] * 6,
    )
    def k(h_hbm, z_hbm, edges_hbm, out_hbm, acc_sh, idx_v, rows_v,
          sem_i0, sem_i1, sem_g0, sem_g1, sem_s0, sem_s1):
        sem_i = (sem_i0, sem_i1)
        sem_g = (sem_g0, sem_g1)
        sem_s = (sem_s0, sem_s1)
        c = lax.axis_index("c")
        s = lax.axis_index("s")
        row0 = s * ROWS_PER_TILE
        rows = pl.ds(row0, ROWS_PER_TILE)

        @pl.when(c == 0)
        def _():
            pltpu.sync_copy(h_hbm.at[rows], acc_sh.at[rows])

        @pl.when(c != 0)
        def _():
            pltpu.sync_copy(z_hbm.at[rows], acc_sh.at[rows])

        chunk0 = (c * NUM_SUBCORES + s) * CHUNKS_PER_TILE

        def wait_idx(kb):
            pltpu.make_async_copy(edges_hbm.at[pl.ds(0, BLKCH)],
                                  idx_v.at[kb], sem_i[kb]).wait()

        def wait_rows(sem):
            pltpu.make_async_copy(h_hbm.at[pl.ds(0, CHUNK)],
                                  rows_v.at[0], sem).wait()

        def fire_gather(kb, jj, slot):
            pltpu.async_copy(h_hbm.at[idx_v.at[kb].at[jj].at[0]],
                             rows_v.at[slot], sem_g[slot])

        def fire_scatter(kb, jj, slot):
            pltpu.async_copy(rows_v.at[slot],
                             acc_sh.at[idx_v.at[kb].at[jj].at[1]],
                             sem_s[slot], add=True)

        def do_pair(kb, jj, last_blk=False):
            wait_rows(sem_g[0])
            fire_scatter(kb, jj, 0)
            wait_rows(sem_g[1])
            fire_scatter(kb, jj + 1, 1)
            if jj == 6 and not last_blk:
                wait_idx(1 - kb)
            wait_rows(sem_s[0])
            if not (last_blk and jj == 6):
                if jj < 6:
                    fire_gather(kb, jj + 2, 0)
                else:
                    fire_gather(1 - kb, 0, 0)
            wait_rows(sem_s[1])
            if not (last_blk and jj == 6):
                if jj < 6:
                    fire_gather(kb, jj + 3, 1)
                else:
                    fire_gather(1 - kb, 1, 1)

        pltpu.sync_copy(edges_hbm.at[pl.ds(chunk0, BLKCH)], idx_v.at[0])
        pltpu.async_copy(edges_hbm.at[pl.ds(chunk0 + BLKCH, BLKCH)],
                         idx_v.at[1], sem_i[1])
        plsc.subcore_barrier()
        fire_gather(0, 0, 0)
        fire_gather(0, 1, 1)

        @pl.loop(0, (NBLOCKS - 2) // 2)
        def _(p):
            bA = 2 * p
            for jj in range(0, BLKCH, 2):
                do_pair(0, jj)
            pltpu.async_copy(
                edges_hbm.at[pl.ds(chunk0 + (bA + 2) * BLKCH, BLKCH)],
                idx_v.at[0], sem_i[0])
            for jj in range(0, BLKCH, 2):
                do_pair(1, jj)
            pltpu.async_copy(
                edges_hbm.at[pl.ds(chunk0 + (bA + 3) * BLKCH, BLKCH)],
                idx_v.at[1], sem_i[1])

        for jj in range(0, BLKCH, 2):
            do_pair(0, jj)
        for jj in range(0, BLKCH, 2):
            do_pair(1, jj, last_blk=True)

        plsc.subcore_barrier()
        pltpu.sync_copy(acc_sh.at[rows], out_hbm.at[c].at[rows])

    return k(h, zeros_f, edges)


def _tc_matmul(x, W):
    def body(x_ref, w_ref, o_ref):
        o_ref[...] = jnp.dot(x_ref[...], w_ref[...],
                             preferred_element_type=jnp.float32)

    return pl.pallas_call(
        body,
        out_shape=jax.ShapeDtypeStruct((x.shape[0], W.shape[1]), jnp.float32),
    )(x, W)


def _tc_scale(cnt, hw):
    def body(cnt_ref, hw_ref, dinv_ref, h_ref):
        deg = 1.0 + cnt_ref[0, :, 0:1] + cnt_ref[1, :, 0:1]
        dinv = lax.rsqrt(deg)
        dinv_ref[...] = dinv
        live = (lax.broadcasted_iota(jnp.int32, (NPAD, 1), 0)
                < N).astype(jnp.float32)
        f = hw_ref.shape[1]
        h_ref[:, 0:f] = live * dinv * hw_ref[...]
        h_ref[:, f:D] = jnp.zeros((NPAD, D - f), jnp.float32)

    return pl.pallas_call(
        body,
        out_shape=(
            jax.ShapeDtypeStruct((NPAD, 1), jnp.float32),
            jax.ShapeDtypeStruct((NPAD, D), jnp.float32),
        ),
    )(cnt, hw)


def _tc_layer(a, dinv, b, W):
    fin = W.shape[0]
    fout = W.shape[1]

    def body(a_ref, dinv_ref, b_ref, w_ref, o_ref):
        h = a_ref[0, :, 0:fin] + a_ref[1, :, 0:fin]
        h = jnp.maximum(dinv_ref[...] * h + b_ref[...], 0.0)
        live = (lax.broadcasted_iota(jnp.int32, (NPAD, 1), 0)
                < N).astype(jnp.float32)
        o_ref[:, 0:fout] = live * dinv_ref[...] * jnp.dot(
            h, w_ref[...], preferred_element_type=jnp.float32)
        if fout < D:
            o_ref[:, fout:D] = jnp.zeros((NPAD, D - fout), jnp.float32)

    return pl.pallas_call(
        body,
        out_shape=jax.ShapeDtypeStruct((NPAD, D), jnp.float32),
    )(a, dinv, b, W)


def _tc_head(a, dinv, b3, batch2, Wmu, bmu, Wlv, blv, eps, Wd1, bd1, gamma,
             beta, Wd2, bd2):
    def body(a_ref, dinv_ref, b3_ref, batch_ref, wmu_ref, bmu_ref, wlv_ref,
             blv_ref, eps_ref, wd1_ref, bd1_ref, g_ref, be_ref, wd2_ref,
             bd2_ref, mu_ref, lv_ref, d2_ref):
        h = a_ref[0] + a_ref[1]
        h = jnp.maximum(dinv_ref[...] * h + b3_ref[...], 0.0)
        gids = lax.broadcasted_iota(jnp.int32, (GPOOL, NPAD), 0)
        mask = (gids == batch_ref[...]).astype(jnp.float32)
        sums = jnp.dot(mask, h, preferred_element_type=jnp.float32)
        cnt = jnp.sum(mask, axis=1, keepdims=True)
        pooled = (sums / jnp.maximum(cnt, 1.0))[0:G]
        mu = jnp.dot(pooled, wmu_ref[...],
                     preferred_element_type=jnp.float32) + bmu_ref[...]
        lv = jnp.dot(pooled, wlv_ref[...],
                     preferred_element_type=jnp.float32) + blv_ref[...]
        mu_ref[...] = mu
        lv_ref[...] = lv
        z = mu + eps_ref[...] * jnp.exp(0.5 * lv)
        d = jnp.maximum(jnp.dot(z, wd1_ref[...],
                                preferred_element_type=jnp.float32)
                        + bd1_ref[...], 0.0)
        m = jnp.mean(d, axis=0, keepdims=True)
        v = jnp.mean((d - m) * (d - m), axis=0, keepdims=True)
        dn = (d - m) / jnp.sqrt(v + 1e-5) * g_ref[...] + be_ref[...]
        dn = jnp.maximum(dn, 0.0)
        d2_ref[...] = jnp.dot(dn, wd2_ref[...],
                              preferred_element_type=jnp.float32) + bd2_ref[...]

    return pl.pallas_call(
        body,
        out_shape=(
            jax.ShapeDtypeStruct((G, LAT), jnp.float32),
            jax.ShapeDtypeStruct((G, LAT), jnp.float32),
            jax.ShapeDtypeStruct((G, MAXN * 32), jnp.float32),
        ),
    )(a, dinv, b3, batch2, Wmu, bmu, Wlv, blv, eps, Wd1, bd1, gamma, beta,
      Wd2, bd2)


def _tc_adj(nr):
    def body(nr_ref, o_ref):
        v = nr_ref[0]
        a = lax.dot_general(v, v, dimension_numbers=(((1,), (1,)), ((), ())),
                            preferred_element_type=jnp.float32)
        r = lax.broadcasted_iota(jnp.int32, (MAXN, MAXN), 0)
        cc = lax.broadcasted_iota(jnp.int32, (MAXN, MAXN), 1)
        a = jnp.where(r == cc, 0.0, a)
        o_ref[0] = 1.0 / (1.0 + jnp.exp(-a))

    return pl.pallas_call(
        body,
        grid=(G,),
        in_specs=[pl.BlockSpec((1, MAXN, 32), lambda g: (g, 0, 0))],
        out_specs=pl.BlockSpec((1, MAXN, MAXN), lambda g: (g, 0, 0)),
        out_shape=jax.ShapeDtypeStruct((G, MAXN, MAXN), jnp.float32),
    )(nr)


def kernel(x, edge_index, batch, W1, b1, W2, b2, W3, b3, Wmu, bmu, Wlv, blv,
           Wd1, bd1, gamma, beta, Wd2, bd2):
    f32 = jnp.float32
    xp = jnp.pad(x, ((0, NPAD - N), (0, 0)))
    fill = N + (jnp.arange(EPAD - E, dtype=jnp.int32) % (NPAD - N))
    srcp = jnp.concatenate([edge_index[0], fill])
    dstp = jnp.concatenate([edge_index[1], fill])
    edges = jnp.stack([srcp.reshape(NCHUNKS, CHUNK),
                       dstp.reshape(NCHUNKS, CHUNK)], axis=1)

    zeros_cnt = jnp.zeros((NPAD, D), f32)
    ones_blk = jnp.ones((CHUNK, D), f32)
    z128 = jnp.zeros((NPAD, D), f32)

    cnt = _sc_count(edges, zeros_cnt, ones_blk)
    hw1 = _tc_matmul(xp, W1)
    dinv, h1p = _tc_scale(cnt, hw1)

    a1 = _sc_aggregate(h1p, z128, edges)
    h2p = _tc_layer(a1, dinv, b1.reshape(1, -1), W2)
    a2 = _sc_aggregate(h2p, z128, edges)
    h3p = _tc_layer(a2, dinv, b2.reshape(1, -1), W3)
    a3 = _sc_aggregate(h3p, z128, edges)

    eps = jax.random.normal(jax.random.key(42), (G, LAT), f32)
    batch2 = jnp.pad(batch, (0, NPAD - N), constant_values=G).reshape(1, NPAD)
    mu, logvar, d2 = _tc_head(
        a3, dinv, b3.reshape(1, -1), batch2, Wmu, bmu.reshape(1, -1), Wlv,
        blv.reshape(1, -1), eps, Wd1, bd1.reshape(1, -1),
        gamma.reshape(1, -1), beta.reshape(1, -1), Wd2, bd2.reshape(1, -1))

    nr = d2.reshape(G, MAXN, 32)
    adj = _tc_adj(nr)
    return adj, mu, logvar


# speedup vs baseline: 3.1712x; 1.1689x over previous
import functools

import jax
import jax.numpy as jnp
from jax import lax
from jax.experimental import pallas as pl
from jax.experimental.pallas import tpu as pltpu
from jax.experimental.pallas import tpu_sc as plsc

N = 10000
D = 128
G = 20
MAXN = 500
LAT = 64
E = 320000

NUM_CORES = 2
NUM_SUBCORES = 16
NUM_TILES = NUM_CORES * NUM_SUBCORES

ROWS_PER_TILE = 632
NPAD = NUM_SUBCORES * ROWS_PER_TILE
SRC_FILL = NPAD - 2
DST_FILL = NPAD - 1

EDGES_PER_TILE = 10240
EPAD = NUM_TILES * EDGES_PER_TILE

CHUNK = 128
CHUNKS_PER_TILE = EDGES_PER_TILE // CHUNK
NCHUNKS = EPAD // CHUNK
NBUF = 4

CHUNKA = 64
CPT_A = EDGES_PER_TILE // CHUNKA
NCHUNKS_A = EPAD // CHUNKA
NSLOT = 4
BLKCH = 8
NBLOCKS = CPT_A // BLKCH

GPOOL = 32


def _sc_mesh():
    return plsc.VectorSubcoreMesh(core_axis_name="c", subcore_axis_name="s")


def _sc_count(edges, zeros_cnt, ones_blk):
    @functools.partial(
        pl.kernel,
        out_type=jax.ShapeDtypeStruct((NUM_CORES, NPAD, D), jnp.float32),
        mesh=_sc_mesh(),
        scratch_types=[
            pltpu.VMEM_SHARED((NPAD, D), jnp.float32),
            pltpu.VMEM((CHUNKS_PER_TILE, 2, CHUNK), jnp.int32),
            pltpu.VMEM((CHUNK, D), jnp.float32),
        ] + [pltpu.SemaphoreType.DMA] * NBUF,
    )
    def k(edges_hbm, zeros_hbm, ones_hbm, out_hbm, acc_sh, idx_v, ones_v,
          *sem_s):
        c = lax.axis_index("c")
        s = lax.axis_index("s")
        row0 = s * ROWS_PER_TILE
        rows = pl.ds(row0, ROWS_PER_TILE)
        pltpu.sync_copy(zeros_hbm.at[rows], acc_sh.at[rows])
        pltpu.sync_copy(ones_hbm, ones_v)
        chunk0 = (c * NUM_SUBCORES + s) * CHUNKS_PER_TILE
        pltpu.sync_copy(edges_hbm.at[pl.ds(chunk0, CHUNKS_PER_TILE)], idx_v)
        plsc.subcore_barrier()

        for b in range(NBUF):
            pltpu.async_copy(ones_v, acc_sh.at[idx_v.at[b].at[1]], sem_s[b],
                             add=True)

        @pl.loop(0, CHUNKS_PER_TILE // NBUF - 1)
        def _(t):
            j0 = t * NBUF + NBUF
            for b in range(NBUF):
                pltpu.make_async_copy(zeros_hbm.at[pl.ds(0, CHUNK)], ones_v,
                                      sem_s[b]).wait()
                pltpu.async_copy(ones_v, acc_sh.at[idx_v.at[j0 + b].at[1]],
                                 sem_s[b], add=True)

        for b in range(NBUF):
            pltpu.make_async_copy(zeros_hbm.at[pl.ds(0, CHUNK)], ones_v,
                                  sem_s[b]).wait()

        plsc.subcore_barrier()
        pltpu.sync_copy(acc_sh.at[rows], out_hbm.at[c].at[rows])

    return k(edges, zeros_cnt, ones_blk)


def _sc_aggregate(h, zeros_f, edges):
    @functools.partial(
        pl.kernel,
        out_type=jax.ShapeDtypeStruct((NUM_CORES, NPAD, D), jnp.float32),
        mesh=_sc_mesh(),
        scratch_types=[
            pltpu.VMEM_SHARED((NPAD, D), jnp.float32),
            pltpu.VMEM((2, BLKCH, 2, CHUNKA), jnp.int32),
            pltpu.VMEM((NSLOT, CHUNKA, D), jnp.float32),
        ] + [pltpu.SemaphoreType.DMA] * (2 + 2 * NSLOT),
    )
    def k(h_hbm, z_hbm, edges_hbm, out_hbm, acc_sh, idx_v, rows_v, *sems):
        sem_i = sems[0:2]
        sem_g = sems[2:2 + NSLOT]
        sem_s = sems[2 + NSLOT:]
        c = lax.axis_index("c")
        s = lax.axis_index("s")
        row0 = s * ROWS_PER_TILE
        rows = pl.ds(row0, ROWS_PER_TILE)

        @pl.when(c == 0)
        def _():
            pltpu.sync_copy(h_hbm.at[rows], acc_sh.at[rows])

        @pl.when(c != 0)
        def _():
            pltpu.sync_copy(z_hbm.at[rows], acc_sh.at[rows])

        chunk0 = (c * NUM_SUBCORES + s) * CPT_A

        def wait_idx(kb):
            pltpu.make_async_copy(edges_hbm.at[pl.ds(0, BLKCH)],
                                  idx_v.at[kb], sem_i[kb]).wait()

        def wait_rows(sem):
            pltpu.make_async_copy(h_hbm.at[pl.ds(0, CHUNKA)],
                                  rows_v.at[0], sem).wait()

        def fire_gather(kb, jj, slot):
            pltpu.async_copy(h_hbm.at[idx_v.at[kb].at[jj].at[0]],
                             rows_v.at[slot], sem_g[slot])

        def fire_scatter(kb, jj, slot):
            pltpu.async_copy(rows_v.at[slot],
                             acc_sh.at[idx_v.at[kb].at[jj].at[1]],
                             sem_s[slot], add=True)

        def do_quad(kb, jj, last_blk=False):
            for b in range(NSLOT):
                wait_rows(sem_g[b])
                fire_scatter(kb, jj + b, b)
            if jj == BLKCH - NSLOT and not last_blk:
                wait_idx(1 - kb)
            for b in range(NSLOT):
                wait_rows(sem_s[b])
                if not (last_blk and jj == BLKCH - NSLOT):
                    if jj < BLKCH - NSLOT:
                        fire_gather(kb, jj + NSLOT + b, b)
                    else:
                        fire_gather(1 - kb, b, b)

        pltpu.sync_copy(edges_hbm.at[pl.ds(chunk0, BLKCH)], idx_v.at[0])
        pltpu.async_copy(edges_hbm.at[pl.ds(chunk0 + BLKCH, BLKCH)],
                         idx_v.at[1], sem_i[1])
        plsc.subcore_barrier()
        for b in range(NSLOT):
            fire_gather(0, b, b)

        @pl.loop(0, (NBLOCKS - 2) // 2)
        def _(p):
            bA = 2 * p
            for jj in range(0, BLKCH, NSLOT):
                do_quad(0, jj)
            pltpu.async_copy(
                edges_hbm.at[pl.ds(chunk0 + (bA + 2) * BLKCH, BLKCH)],
                idx_v.at[0], sem_i[0])
            for jj in range(0, BLKCH, NSLOT):
                do_quad(1, jj)
            pltpu.async_copy(
                edges_hbm.at[pl.ds(chunk0 + (bA + 3) * BLKCH, BLKCH)],
                idx_v.at[1], sem_i[1])

        for jj in range(0, BLKCH, NSLOT):
            do_quad(0, jj)
        for jj in range(0, BLKCH, NSLOT):
            do_quad(1, jj, last_blk=(jj == BLKCH - NSLOT))

        plsc.subcore_barrier()
        pltpu.sync_copy(acc_sh.at[rows], out_hbm.at[c].at[rows])

    return k(h, zeros_f, edges)


def _tc_matmul(x, W):
    def body(x_ref, w_ref, o_ref):
        o_ref[...] = jnp.dot(x_ref[...], w_ref[...],
                             preferred_element_type=jnp.float32)

    return pl.pallas_call(
        body,
        out_shape=jax.ShapeDtypeStruct((x.shape[0], W.shape[1]), jnp.float32),
    )(x, W)


def _tc_scale(cnt, hw):
    def body(cnt_ref, hw_ref, dinv_ref, h_ref):
        deg = 1.0 + cnt_ref[0, :, 0:1] + cnt_ref[1, :, 0:1]
        dinv = lax.rsqrt(deg)
        dinv_ref[...] = dinv
        live = (lax.broadcasted_iota(jnp.int32, (NPAD, 1), 0)
                < N).astype(jnp.float32)
        f = hw_ref.shape[1]
        h_ref[:, 0:f] = live * dinv * hw_ref[...]
        h_ref[:, f:D] = jnp.zeros((NPAD, D - f), jnp.float32)

    return pl.pallas_call(
        body,
        out_shape=(
            jax.ShapeDtypeStruct((NPAD, 1), jnp.float32),
            jax.ShapeDtypeStruct((NPAD, D), jnp.float32),
        ),
    )(cnt, hw)


def _tc_layer(a, dinv, b, W):
    fin = W.shape[0]
    fout = W.shape[1]

    def body(a_ref, dinv_ref, b_ref, w_ref, o_ref):
        h = a_ref[0, :, 0:fin] + a_ref[1, :, 0:fin]
        h = jnp.maximum(dinv_ref[...] * h + b_ref[...], 0.0)
        live = (lax.broadcasted_iota(jnp.int32, (NPAD, 1), 0)
                < N).astype(jnp.float32)
        o_ref[:, 0:fout] = live * dinv_ref[...] * jnp.dot(
            h, w_ref[...], preferred_element_type=jnp.float32)
        if fout < D:
            o_ref[:, fout:D] = jnp.zeros((NPAD, D - fout), jnp.float32)

    return pl.pallas_call(
        body,
        out_shape=jax.ShapeDtypeStruct((NPAD, D), jnp.float32),
    )(a, dinv, b, W)


def _tc_head(a, dinv, b3, batch2, Wmu, bmu, Wlv, blv, eps, Wd1, bd1, gamma,
             beta, Wd2, bd2):
    def body(a_ref, dinv_ref, b3_ref, batch_ref, wmu_ref, bmu_ref, wlv_ref,
             blv_ref, eps_ref, wd1_ref, bd1_ref, g_ref, be_ref, wd2_ref,
             bd2_ref, mu_ref, lv_ref, d2_ref):
        h = a_ref[0] + a_ref[1]
        h = jnp.maximum(dinv_ref[...] * h + b3_ref[...], 0.0)
        gids = lax.broadcasted_iota(jnp.int32, (GPOOL, NPAD), 0)
        mask = (gids == batch_ref[...]).astype(jnp.float32)
        sums = jnp.dot(mask, h, preferred_element_type=jnp.float32)
        cnt = jnp.sum(mask, axis=1, keepdims=True)
        pooled = (sums / jnp.maximum(cnt, 1.0))[0:G]
        mu = jnp.dot(pooled, wmu_ref[...],
                     preferred_element_type=jnp.float32) + bmu_ref[...]
        lv = jnp.dot(pooled, wlv_ref[...],
                     preferred_element_type=jnp.float32) + blv_ref[...]
        mu_ref[...] = mu
        lv_ref[...] = lv
        z = mu + eps_ref[...] * jnp.exp(0.5 * lv)
        d = jnp.maximum(jnp.dot(z, wd1_ref[...],
                                preferred_element_type=jnp.float32)
                        + bd1_ref[...], 0.0)
        m = jnp.mean(d, axis=0, keepdims=True)
        v = jnp.mean((d - m) * (d - m), axis=0, keepdims=True)
        dn = (d - m) / jnp.sqrt(v + 1e-5) * g_ref[...] + be_ref[...]
        dn = jnp.maximum(dn, 0.0)
        d2_ref[...] = jnp.dot(dn, wd2_ref[...],
                              preferred_element_type=jnp.float32) + bd2_ref[...]

    return pl.pallas_call(
        body,
        out_shape=(
            jax.ShapeDtypeStruct((G, LAT), jnp.float32),
            jax.ShapeDtypeStruct((G, LAT), jnp.float32),
            jax.ShapeDtypeStruct((G, MAXN * 32), jnp.float32),
        ),
    )(a, dinv, b3, batch2, Wmu, bmu, Wlv, blv, eps, Wd1, bd1, gamma, beta,
      Wd2, bd2)


def _tc_adj(nr):
    def body(nr_ref, o_ref):
        v = nr_ref[0]
        a = lax.dot_general(v, v, dimension_numbers=(((1,), (1,)), ((), ())),
                            preferred_element_type=jnp.float32)
        r = lax.broadcasted_iota(jnp.int32, (MAXN, MAXN), 0)
        cc = lax.broadcasted_iota(jnp.int32, (MAXN, MAXN), 1)
        a = jnp.where(r == cc, 0.0, a)
        o_ref[0] = 1.0 / (1.0 + jnp.exp(-a))

    return pl.pallas_call(
        body,
        grid=(G,),
        in_specs=[pl.BlockSpec((1, MAXN, 32), lambda g: (g, 0, 0))],
        out_specs=pl.BlockSpec((1, MAXN, MAXN), lambda g: (g, 0, 0)),
        out_shape=jax.ShapeDtypeStruct((G, MAXN, MAXN), jnp.float32),
    )(nr)


def kernel(x, edge_index, batch, W1, b1, W2, b2, W3, b3, Wmu, bmu, Wlv, blv,
           Wd1, bd1, gamma, beta, Wd2, bd2):
    f32 = jnp.float32
    xp = jnp.pad(x, ((0, NPAD - N), (0, 0)))
    fill = N + (jnp.arange(EPAD - E, dtype=jnp.int32) % (NPAD - N))
    srcp = jnp.concatenate([edge_index[0], fill])
    dstp = jnp.concatenate([edge_index[1], fill])
    edges = jnp.stack([srcp.reshape(NCHUNKS, CHUNK),
                       dstp.reshape(NCHUNKS, CHUNK)], axis=1)
    edges_a = jnp.stack([srcp.reshape(NCHUNKS_A, CHUNKA),
                         dstp.reshape(NCHUNKS_A, CHUNKA)], axis=1)

    zeros_cnt = jnp.zeros((NPAD, D), f32)
    ones_blk = jnp.ones((CHUNK, D), f32)
    z128 = jnp.zeros((NPAD, D), f32)

    cnt = _sc_count(edges, zeros_cnt, ones_blk)
    hw1 = _tc_matmul(xp, W1)
    dinv, h1p = _tc_scale(cnt, hw1)

    a1 = _sc_aggregate(h1p, z128, edges_a)
    h2p = _tc_layer(a1, dinv, b1.reshape(1, -1), W2)
    a2 = _sc_aggregate(h2p, z128, edges_a)
    h3p = _tc_layer(a2, dinv, b2.reshape(1, -1), W3)
    a3 = _sc_aggregate(h3p, z128, edges_a)

    eps = jax.random.normal(jax.random.key(42), (G, LAT), f32)
    batch2 = jnp.pad(batch, (0, NPAD - N), constant_values=G).reshape(1, NPAD)
    mu, logvar, d2 = _tc_head(
        a3, dinv, b3.reshape(1, -1), batch2, Wmu, bmu.reshape(1, -1), Wlv,
        blv.reshape(1, -1), eps, Wd1, bd1.reshape(1, -1),
        gamma.reshape(1, -1), beta.reshape(1, -1), Wd2, bd2.reshape(1, -1))

    nr = d2.reshape(G, MAXN, 32)
    adj = _tc_adj(nr)
    return adj, mu, logvar


# speedup vs baseline: 3.5904x; 1.1322x over previous
import functools

import jax
import jax.numpy as jnp
from jax import lax
from jax.experimental import pallas as pl
from jax.experimental.pallas import tpu as pltpu
from jax.experimental.pallas import tpu_sc as plsc

N = 10000
D = 128
G = 20
MAXN = 500
LAT = 64
E = 320000

NUM_CORES = 2
NUM_SUBCORES = 16
NUM_TILES = NUM_CORES * NUM_SUBCORES

ROWS_PER_TILE = 632
NPAD = NUM_SUBCORES * ROWS_PER_TILE
SRC_FILL = NPAD - 2
DST_FILL = NPAD - 1

EDGES_PER_TILE = 10240
EPAD = NUM_TILES * EDGES_PER_TILE

CHUNK = 128
CHUNKS_PER_TILE = EDGES_PER_TILE // CHUNK
NCHUNKS = EPAD // CHUNK
NBUF = 4

CHUNKA = 64
CPT_A = EDGES_PER_TILE // CHUNKA
NCHUNKS_A = EPAD // CHUNKA
NSLOT = 4
BLKCH = 8
NBLOCKS = CPT_A // BLKCH

GPOOL = 32


def _sc_mesh():
    return plsc.VectorSubcoreMesh(core_axis_name="c", subcore_axis_name="s")


def _sc_count(edges, zeros_cnt, ones_blk):
    @functools.partial(
        pl.kernel,
        out_type=jax.ShapeDtypeStruct((NUM_CORES, NPAD, D), jnp.float32),
        mesh=_sc_mesh(),
        scratch_types=[
            pltpu.VMEM_SHARED((NPAD, D), jnp.float32),
            pltpu.VMEM((CHUNKS_PER_TILE, 2, CHUNK), jnp.int32),
            pltpu.VMEM((CHUNK, D), jnp.float32),
        ] + [pltpu.SemaphoreType.DMA] * NBUF,
    )
    def k(edges_hbm, zeros_hbm, ones_hbm, out_hbm, acc_sh, idx_v, ones_v,
          *sem_s):
        c = lax.axis_index("c")
        s = lax.axis_index("s")
        row0 = s * ROWS_PER_TILE
        rows = pl.ds(row0, ROWS_PER_TILE)
        pltpu.sync_copy(zeros_hbm.at[rows], acc_sh.at[rows])
        pltpu.sync_copy(ones_hbm, ones_v)
        chunk0 = (c * NUM_SUBCORES + s) * CHUNKS_PER_TILE
        pltpu.sync_copy(edges_hbm.at[pl.ds(chunk0, CHUNKS_PER_TILE)], idx_v)
        plsc.subcore_barrier()

        sem_s0 = sem_s[0]
        pltpu.async_copy(ones_v, acc_sh.at[idx_v.at[0].at[1]], sem_s0,
                         add=True)

        @pl.loop(0, CHUNKS_PER_TILE - 1)
        def _(t):
            pltpu.make_async_copy(zeros_hbm.at[pl.ds(0, CHUNK)], ones_v,
                                  sem_s0).wait()
            pltpu.async_copy(ones_v, acc_sh.at[idx_v.at[t + 1].at[1]],
                             sem_s0, add=True)

        pltpu.make_async_copy(zeros_hbm.at[pl.ds(0, CHUNK)], ones_v,
                              sem_s0).wait()

        plsc.subcore_barrier()
        pltpu.sync_copy(acc_sh.at[rows], out_hbm.at[c].at[rows])

    return k(edges, zeros_cnt, ones_blk)


def _sc_aggregate(h, zeros_f, edges):
    @functools.partial(
        pl.kernel,
        out_type=jax.ShapeDtypeStruct((NUM_CORES, NPAD, D), jnp.float32),
        mesh=_sc_mesh(),
        scratch_types=[
            pltpu.VMEM_SHARED((NPAD, D), jnp.float32),
            pltpu.VMEM((2, BLKCH, 2, CHUNKA), jnp.int32),
            pltpu.VMEM((NSLOT, CHUNKA, D), jnp.float32),
        ] + [pltpu.SemaphoreType.DMA] * (3 + NSLOT),
    )
    def k(h_hbm, z_hbm, edges_hbm, out_hbm, acc_sh, idx_v, rows_v, *sems):
        sem_i = sems[0:2]
        sem_g = sems[2:2 + NSLOT]
        sem_s = sems[2 + NSLOT]
        c = lax.axis_index("c")
        s = lax.axis_index("s")
        row0 = s * ROWS_PER_TILE
        rows = pl.ds(row0, ROWS_PER_TILE)

        @pl.when(c == 0)
        def _():
            pltpu.sync_copy(h_hbm.at[rows], acc_sh.at[rows])

        @pl.when(c != 0)
        def _():
            pltpu.sync_copy(z_hbm.at[rows], acc_sh.at[rows])

        chunk0 = (c * NUM_SUBCORES + s) * CPT_A

        def wait_idx(kb):
            pltpu.make_async_copy(edges_hbm.at[pl.ds(0, BLKCH)],
                                  idx_v.at[kb], sem_i[kb]).wait()

        def wait_rows(sem):
            pltpu.make_async_copy(h_hbm.at[pl.ds(0, CHUNKA)],
                                  rows_v.at[0], sem).wait()

        def fire_gather(kb, jj, slot):
            pltpu.async_copy(h_hbm.at[idx_v.at[kb].at[jj].at[0]],
                             rows_v.at[slot], sem_g[slot])

        def fire_scatter(kb, jj, slot):
            pltpu.async_copy(rows_v.at[slot],
                             acc_sh.at[idx_v.at[kb].at[jj].at[1]],
                             sem_s, add=True)

        def do_quad(kb, jj, last_blk=False):
            if jj == BLKCH - NSLOT and not last_blk:
                wait_idx(1 - kb)
            for b in range(NSLOT):
                wait_rows(sem_g[b])
                fire_scatter(kb, jj + b, b)
                wait_rows(sem_s)
                if not (last_blk and jj == BLKCH - NSLOT):
                    if jj < BLKCH - NSLOT:
                        fire_gather(kb, jj + NSLOT + b, b)
                    else:
                        fire_gather(1 - kb, b, b)

        pltpu.sync_copy(edges_hbm.at[pl.ds(chunk0, BLKCH)], idx_v.at[0])
        pltpu.async_copy(edges_hbm.at[pl.ds(chunk0 + BLKCH, BLKCH)],
                         idx_v.at[1], sem_i[1])
        plsc.subcore_barrier()
        for b in range(NSLOT):
            fire_gather(0, b, b)

        @pl.loop(0, (NBLOCKS - 2) // 2)
        def _(p):
            bA = 2 * p
            for jj in range(0, BLKCH, NSLOT):
                do_quad(0, jj)
            pltpu.async_copy(
                edges_hbm.at[pl.ds(chunk0 + (bA + 2) * BLKCH, BLKCH)],
                idx_v.at[0], sem_i[0])
            for jj in range(0, BLKCH, NSLOT):
                do_quad(1, jj)
            pltpu.async_copy(
                edges_hbm.at[pl.ds(chunk0 + (bA + 3) * BLKCH, BLKCH)],
                idx_v.at[1], sem_i[1])

        for jj in range(0, BLKCH, NSLOT):
            do_quad(0, jj)
        for jj in range(0, BLKCH, NSLOT):
            do_quad(1, jj, last_blk=(jj == BLKCH - NSLOT))

        plsc.subcore_barrier()
        pltpu.sync_copy(acc_sh.at[rows], out_hbm.at[c].at[rows])

    return k(h, zeros_f, edges)


def _tc_matmul(x, W):
    def body(x_ref, w_ref, o_ref):
        o_ref[...] = jnp.dot(x_ref[...], w_ref[...],
                             preferred_element_type=jnp.float32)

    return pl.pallas_call(
        body,
        out_shape=jax.ShapeDtypeStruct((x.shape[0], W.shape[1]), jnp.float32),
    )(x, W)


def _tc_scale(cnt, hw):
    def body(cnt_ref, hw_ref, dinv_ref, h_ref):
        deg = 1.0 + cnt_ref[0, :, 0:1] + cnt_ref[1, :, 0:1]
        dinv = lax.rsqrt(deg)
        dinv_ref[...] = dinv
        live = (lax.broadcasted_iota(jnp.int32, (NPAD, 1), 0)
                < N).astype(jnp.float32)
        f = hw_ref.shape[1]
        h_ref[:, 0:f] = live * dinv * hw_ref[...]
        h_ref[:, f:D] = jnp.zeros((NPAD, D - f), jnp.float32)

    return pl.pallas_call(
        body,
        out_shape=(
            jax.ShapeDtypeStruct((NPAD, 1), jnp.float32),
            jax.ShapeDtypeStruct((NPAD, D), jnp.float32),
        ),
    )(cnt, hw)


def _tc_layer(a, dinv, b, W):
    fin = W.shape[0]
    fout = W.shape[1]

    def body(a_ref, dinv_ref, b_ref, w_ref, o_ref):
        h = a_ref[0, :, 0:fin] + a_ref[1, :, 0:fin]
        h = jnp.maximum(dinv_ref[...] * h + b_ref[...], 0.0)
        live = (lax.broadcasted_iota(jnp.int32, (NPAD, 1), 0)
                < N).astype(jnp.float32)
        o_ref[:, 0:fout] = live * dinv_ref[...] * jnp.dot(
            h, w_ref[...], preferred_element_type=jnp.float32)
        if fout < D:
            o_ref[:, fout:D] = jnp.zeros((NPAD, D - fout), jnp.float32)

    return pl.pallas_call(
        body,
        out_shape=jax.ShapeDtypeStruct((NPAD, D), jnp.float32),
    )(a, dinv, b, W)


def _tc_head(a, dinv, b3, batch2, Wmu, bmu, Wlv, blv, eps, Wd1, bd1, gamma,
             beta, Wd2, bd2):
    def body(a_ref, dinv_ref, b3_ref, batch_ref, wmu_ref, bmu_ref, wlv_ref,
             blv_ref, eps_ref, wd1_ref, bd1_ref, g_ref, be_ref, wd2_ref,
             bd2_ref, mu_ref, lv_ref, d2_ref):
        h = a_ref[0] + a_ref[1]
        h = jnp.maximum(dinv_ref[...] * h + b3_ref[...], 0.0)
        gids = lax.broadcasted_iota(jnp.int32, (GPOOL, NPAD), 0)
        mask = (gids == batch_ref[...]).astype(jnp.float32)
        sums = jnp.dot(mask, h, preferred_element_type=jnp.float32)
        cnt = jnp.sum(mask, axis=1, keepdims=True)
        pooled = (sums / jnp.maximum(cnt, 1.0))[0:G]
        mu = jnp.dot(pooled, wmu_ref[...],
                     preferred_element_type=jnp.float32) + bmu_ref[...]
        lv = jnp.dot(pooled, wlv_ref[...],
                     preferred_element_type=jnp.float32) + blv_ref[...]
        mu_ref[...] = mu
        lv_ref[...] = lv
        z = mu + eps_ref[...] * jnp.exp(0.5 * lv)
        d = jnp.maximum(jnp.dot(z, wd1_ref[...],
                                preferred_element_type=jnp.float32)
                        + bd1_ref[...], 0.0)
        m = jnp.mean(d, axis=0, keepdims=True)
        v = jnp.mean((d - m) * (d - m), axis=0, keepdims=True)
        dn = (d - m) / jnp.sqrt(v + 1e-5) * g_ref[...] + be_ref[...]
        dn = jnp.maximum(dn, 0.0)
        d2_ref[...] = jnp.dot(dn, wd2_ref[...],
                              preferred_element_type=jnp.float32) + bd2_ref[...]

    return pl.pallas_call(
        body,
        out_shape=(
            jax.ShapeDtypeStruct((G, LAT), jnp.float32),
            jax.ShapeDtypeStruct((G, LAT), jnp.float32),
            jax.ShapeDtypeStruct((G, MAXN * 32), jnp.float32),
        ),
    )(a, dinv, b3, batch2, Wmu, bmu, Wlv, blv, eps, Wd1, bd1, gamma, beta,
      Wd2, bd2)


def _tc_adj(nr):
    def body(nr_ref, o_ref):
        v = nr_ref[0]
        a = lax.dot_general(v, v, dimension_numbers=(((1,), (1,)), ((), ())),
                            preferred_element_type=jnp.float32)
        r = lax.broadcasted_iota(jnp.int32, (MAXN, MAXN), 0)
        cc = lax.broadcasted_iota(jnp.int32, (MAXN, MAXN), 1)
        a = jnp.where(r == cc, 0.0, a)
        o_ref[0] = 1.0 / (1.0 + jnp.exp(-a))

    return pl.pallas_call(
        body,
        grid=(G,),
        in_specs=[pl.BlockSpec((1, MAXN, 32), lambda g: (g, 0, 0))],
        out_specs=pl.BlockSpec((1, MAXN, MAXN), lambda g: (g, 0, 0)),
        out_shape=jax.ShapeDtypeStruct((G, MAXN, MAXN), jnp.float32),
    )(nr)


def kernel(x, edge_index, batch, W1, b1, W2, b2, W3, b3, Wmu, bmu, Wlv, blv,
           Wd1, bd1, gamma, beta, Wd2, bd2):
    f32 = jnp.float32
    xp = jnp.pad(x, ((0, NPAD - N), (0, 0)))
    fill = N + (jnp.arange(EPAD - E, dtype=jnp.int32) % (NPAD - N))
    srcp = jnp.concatenate([edge_index[0], fill])
    dstp = jnp.concatenate([edge_index[1], fill])
    edges = jnp.stack([srcp.reshape(NCHUNKS, CHUNK),
                       dstp.reshape(NCHUNKS, CHUNK)], axis=1)
    edges_a = jnp.stack([srcp.reshape(NCHUNKS_A, CHUNKA),
                         dstp.reshape(NCHUNKS_A, CHUNKA)], axis=1)

    zeros_cnt = jnp.zeros((NPAD, D), f32)
    ones_blk = jnp.ones((CHUNK, D), f32)
    z128 = jnp.zeros((NPAD, D), f32)

    cnt = _sc_count(edges, zeros_cnt, ones_blk)
    hw1 = _tc_matmul(xp, W1)
    dinv, h1p = _tc_scale(cnt, hw1)

    a1 = _sc_aggregate(h1p, z128, edges_a)
    h2p = _tc_layer(a1, dinv, b1.reshape(1, -1), W2)
    a2 = _sc_aggregate(h2p, z128, edges_a)
    h3p = _tc_layer(a2, dinv, b2.reshape(1, -1), W3)
    a3 = _sc_aggregate(h3p, z128, edges_a)

    eps = jax.random.normal(jax.random.key(42), (G, LAT), f32)
    batch2 = jnp.pad(batch, (0, NPAD - N), constant_values=G).reshape(1, NPAD)
    mu, logvar, d2 = _tc_head(
        a3, dinv, b3.reshape(1, -1), batch2, Wmu, bmu.reshape(1, -1), Wlv,
        blv.reshape(1, -1), eps, Wd1, bd1.reshape(1, -1),
        gamma.reshape(1, -1), beta.reshape(1, -1), Wd2, bd2.reshape(1, -1))

    nr = d2.reshape(G, MAXN, 32)
    adj = _tc_adj(nr)
    return adj, mu, logvar
